# baseline (speedup 1.0000x reference)
import jax
import jax.numpy as jnp
from jax.experimental import pallas as pl
from jax.experimental.pallas import tpu as pltpu

N = 8192
NFEAT = 128
NHID = 64
NCLASS = 16
ALPHA = 0.2
BLK = 256


def _gcn_kernel(x_ref, adj_ref, W1_ref, b1_ref, W2_ref, b2_ref,
                out_ref, s1_ref, s2_ref):
    phase = pl.program_id(0)
    i = pl.program_id(1)

    @pl.when(jnp.logical_and(phase == 0, i == 0))
    def _():
        s1_ref[...] = jnp.dot(x_ref[...], W1_ref[...],
                              preferred_element_type=jnp.float32)

    @pl.when(phase == 0)
    def _():
        h1 = jnp.dot(adj_ref[...], s1_ref[...],
                     preferred_element_type=jnp.float32) + b1_ref[...]
        h1 = jnp.where(h1 > 0, h1, ALPHA * h1)
        s2_ref[pl.ds(i * BLK, BLK), :] = jnp.dot(
            h1, W2_ref[...], preferred_element_type=jnp.float32)

    @pl.when(phase == 1)
    def _():
        h2 = jnp.dot(adj_ref[...], s2_ref[...],
                     preferred_element_type=jnp.float32) + b2_ref[...]
        m = jnp.max(h2, axis=1, keepdims=True)
        e = jnp.exp(h2 - m)
        lse = jnp.log(jnp.sum(e, axis=1, keepdims=True))
        out_ref[...] = h2 - m - lse


def kernel(x, adj, W1, b1, W2, b2):
    b1r = b1.reshape(1, NHID)
    b2r = b2.reshape(1, NCLASS)
    grid = (2, N // BLK)
    return pl.pallas_call(
        _gcn_kernel,
        grid=grid,
        in_specs=[
            pl.BlockSpec((N, NFEAT), lambda p, i: (0, 0)),
            pl.BlockSpec((BLK, N), lambda p, i: (i, 0)),
            pl.BlockSpec((NFEAT, NHID), lambda p, i: (0, 0)),
            pl.BlockSpec((1, NHID), lambda p, i: (0, 0)),
            pl.BlockSpec((NHID, NCLASS), lambda p, i: (0, 0)),
            pl.BlockSpec((1, NCLASS), lambda p, i: (0, 0)),
        ],
        out_specs=pl.BlockSpec((BLK, NCLASS), lambda p, i: (i, 0)),
        out_shape=jax.ShapeDtypeStruct((N, NCLASS), jnp.float32),
        scratch_shapes=[
            pltpu.VMEM((N, NHID), jnp.float32),
            pltpu.VMEM((N, NCLASS), jnp.float32),
        ],
        compiler_params=pltpu.CompilerParams(
            dimension_semantics=("arbitrary", "arbitrary"),
        ),
    )(x, adj, W1, b1r, W2, b2r)


# speedup vs baseline: 1.0140x; 1.0140x over previous
import jax
import jax.numpy as jnp
from jax.experimental import pallas as pl
from jax.experimental.pallas import tpu as pltpu

N = 8192
NFEAT = 128
NHID = 64
NCLASS = 16
ALPHA = 0.2
BLK = 512


def _gcn_kernel(x_ref, adj_ref, W1_ref, b1_ref, W2_ref, b2_ref,
                out_ref, s1_ref, s2_ref):
    phase = pl.program_id(0)
    i = pl.program_id(1)

    @pl.when(jnp.logical_and(phase == 0, i == 0))
    def _():
        s1_ref[...] = jnp.dot(x_ref[...], W1_ref[...],
                              preferred_element_type=jnp.float32)

    @pl.when(phase == 0)
    def _():
        h1 = jnp.dot(adj_ref[...], s1_ref[...],
                     preferred_element_type=jnp.float32) + b1_ref[...]
        h1 = jnp.where(h1 > 0, h1, ALPHA * h1)
        s2_ref[pl.ds(i * BLK, BLK), :] = jnp.dot(
            h1, W2_ref[...], preferred_element_type=jnp.float32)

    @pl.when(phase == 1)
    def _():
        h2 = jnp.dot(adj_ref[...], s2_ref[...],
                     preferred_element_type=jnp.float32) + b2_ref[...]
        m = jnp.max(h2, axis=1, keepdims=True)
        e = jnp.exp(h2 - m)
        lse = jnp.log(jnp.sum(e, axis=1, keepdims=True))
        out_ref[...] = h2 - m - lse


def kernel(x, adj, W1, b1, W2, b2):
    b1r = b1.reshape(1, NHID)
    b2r = b2.reshape(1, NCLASS)
    grid = (2, N // BLK)
    return pl.pallas_call(
        _gcn_kernel,
        grid=grid,
        in_specs=[
            pl.BlockSpec((N, NFEAT), lambda p, i: (0, 0)),
            pl.BlockSpec((BLK, N), lambda p, i: (i, 0)),
            pl.BlockSpec((NFEAT, NHID), lambda p, i: (0, 0)),
            pl.BlockSpec((1, NHID), lambda p, i: (0, 0)),
            pl.BlockSpec((NHID, NCLASS), lambda p, i: (0, 0)),
            pl.BlockSpec((1, NCLASS), lambda p, i: (0, 0)),
        ],
        out_specs=pl.BlockSpec((BLK, NCLASS), lambda p, i: (i, 0)),
        out_shape=jax.ShapeDtypeStruct((N, NCLASS), jnp.float32),
        scratch_shapes=[
            pltpu.VMEM((N, NHID), jnp.float32),
            pltpu.VMEM((N, NCLASS), jnp.float32),
        ],
        compiler_params=pltpu.CompilerParams(
            dimension_semantics=("arbitrary", "arbitrary"),
        ),
    )(x, adj, W1, b1r, W2, b2r)
